# baseline (speedup 1.0000x reference)
import functools

import jax
import jax.numpy as jnp
from jax import lax
from jax.experimental import pallas as pl
from jax.experimental.pallas import tpu as pltpu
from jax.experimental.pallas import tpu_sc as plsc

H, W, D = 32, 32, 768
B = 32
DH = D // 2
L = 16
NC, NS = 2, 16
HH = H // 2


def _sc_lookup(scale_hbm, row_hbm, col_hbm, tabs_hbm, sbuf, buf, sem):
    cid = lax.axis_index("c")
    sid = lax.axis_index("s")
    wid = cid * NS + sid
    cp_s = pltpu.make_async_copy(scale_hbm, sbuf, sem)
    cp_r = pltpu.make_async_copy(row_hbm.at[wid], buf.at[0], sem)
    cp_c = pltpu.make_async_copy(col_hbm.at[wid], buf.at[1], sem)
    cp_s.start()
    cp_r.start()
    cp_c.start()
    cp_s.wait()
    cp_r.wait()
    cp_c.wait()
    s = sbuf[...]
    for j in range(2):
        for k in range(DH // L):
            buf[j, pl.ds(L * k, L)] = buf[j, pl.ds(L * k, L)] * s
    o_r = pltpu.make_async_copy(buf.at[0], tabs_hbm.at[wid], sem)
    o_c = pltpu.make_async_copy(buf.at[1], tabs_hbm.at[H + wid], sem)
    o_r.start()
    o_c.start()
    o_r.wait()
    o_c.wait()


def _tc_replicate(tabs_ref, out_ref, tile_ref, sems):
    r = tabs_ref[pl.ds(0, H), :]
    c = tabs_ref[pl.ds(H, W), :]
    tile_ref[:, :, :DH] = jnp.broadcast_to(r[:, None, :], (H, W, DH))
    tile_ref[:, :, DH:] = jnp.broadcast_to(c[None, :, :], (H, W, DH))
    copies = []
    for b in range(B):
        for i in range(2):
            copies.append(pltpu.make_async_copy(
                tile_ref.at[pl.ds(i * HH, HH)],
                out_ref.at[b, pl.ds(i * HH, HH)],
                sems.at[2 * b + i]))
    for cp in copies:
        cp.start()
    for cp in copies:
        cp.wait()


def kernel(row_embed, col_embed, batch_size):
    scale = (jnp.asarray(batch_size, jnp.int32) // B).astype(jnp.float32)
    scale_vec = jnp.full((L,), scale, dtype=jnp.float32)
    mesh = plsc.VectorSubcoreMesh(core_axis_name="c", subcore_axis_name="s")
    lookup = functools.partial(
        pl.kernel,
        mesh=mesh,
        out_type=jax.ShapeDtypeStruct((H + W, DH), jnp.float32),
        scratch_types=[
            pltpu.VMEM((L,), jnp.float32),
            pltpu.VMEM((2, DH), jnp.float32),
            pltpu.SemaphoreType.DMA,
        ],
    )(_sc_lookup)
    tabs = lookup(scale_vec, row_embed, col_embed)
    return pl.pallas_call(
        _tc_replicate,
        in_specs=[pl.BlockSpec(memory_space=pltpu.VMEM)],
        out_specs=pl.BlockSpec(memory_space=pl.ANY),
        out_shape=jax.ShapeDtypeStruct((B, H, W, D), jnp.float32),
        scratch_shapes=[
            pltpu.VMEM((H, W, D), jnp.float32),
            pltpu.SemaphoreType.DMA((2 * B,)),
        ],
    )(tabs)


# speedup vs baseline: 1.0075x; 1.0075x over previous
import functools

import jax
import jax.numpy as jnp
from jax import lax
from jax.experimental import pallas as pl
from jax.experimental.pallas import tpu as pltpu
from jax.experimental.pallas import tpu_sc as plsc

H, W, D = 32, 32, 768
B = 32
DH = D // 2
L = 16
NC, NS = 2, 16
HH = H // 2


def _sc_lookup(scale_hbm, row_hbm, col_hbm, tabs_hbm, sbuf, buf, sem):
    cid = lax.axis_index("c")
    sid = lax.axis_index("s")
    wid = cid * NS + sid
    cp_s = pltpu.make_async_copy(scale_hbm, sbuf, sem)
    cp_r = pltpu.make_async_copy(row_hbm.at[wid], buf.at[0], sem)
    cp_c = pltpu.make_async_copy(col_hbm.at[wid], buf.at[1], sem)
    cp_s.start()
    cp_r.start()
    cp_c.start()
    cp_s.wait()
    cp_r.wait()
    cp_c.wait()
    s = sbuf[...]
    for j in range(2):
        for k in range(DH // L):
            buf[j, pl.ds(L * k, L)] = buf[j, pl.ds(L * k, L)] * s
    o_r = pltpu.make_async_copy(buf.at[0], tabs_hbm.at[wid], sem)
    o_c = pltpu.make_async_copy(buf.at[1], tabs_hbm.at[H + wid], sem)
    o_r.start()
    o_c.start()
    o_r.wait()
    o_c.wait()


def _tc_replicate(tabs_ref, out_ref, tile_ref, sems):
    c = tabs_ref[pl.ds(H, W), :]
    copies = []
    for i in range(2):
        r = tabs_ref[pl.ds(i * HH, HH), :]
        tile_ref[pl.ds(i * HH, HH), :, :DH] = jnp.broadcast_to(
            r[:, None, :], (HH, W, DH))
        tile_ref[pl.ds(i * HH, HH), :, DH:] = jnp.broadcast_to(
            c[None, :, :], (HH, W, DH))
        for b in range(B):
            cp = pltpu.make_async_copy(
                tile_ref.at[pl.ds(i * HH, HH)],
                out_ref.at[b, pl.ds(i * HH, HH)],
                sems.at[2 * b + i])
            cp.start()
            copies.append(cp)
    for cp in copies:
        cp.wait()


def kernel(row_embed, col_embed, batch_size):
    scale = (jnp.asarray(batch_size, jnp.int32) // B).astype(jnp.float32)
    scale_vec = jnp.full((L,), scale, dtype=jnp.float32)
    mesh = plsc.VectorSubcoreMesh(core_axis_name="c", subcore_axis_name="s")
    lookup = functools.partial(
        pl.kernel,
        mesh=mesh,
        out_type=jax.ShapeDtypeStruct((H + W, DH), jnp.float32),
        scratch_types=[
            pltpu.VMEM((L,), jnp.float32),
            pltpu.VMEM((2, DH), jnp.float32),
            pltpu.SemaphoreType.DMA,
        ],
    )(_sc_lookup)
    tabs = lookup(scale_vec, row_embed, col_embed)
    return pl.pallas_call(
        _tc_replicate,
        in_specs=[pl.BlockSpec(memory_space=pltpu.VMEM)],
        out_specs=pl.BlockSpec(memory_space=pl.ANY),
        out_shape=jax.ShapeDtypeStruct((B, H, W, D), jnp.float32),
        scratch_shapes=[
            pltpu.VMEM((H, W, D), jnp.float32),
            pltpu.SemaphoreType.DMA((2 * B,)),
        ],
    )(tabs)


# speedup vs baseline: 1.0447x; 1.0369x over previous
import functools

import jax
import jax.numpy as jnp
from jax import lax
from jax.experimental import pallas as pl
from jax.experimental.pallas import tpu as pltpu
from jax.experimental.pallas import tpu_sc as plsc

H, W, D = 32, 32, 768
B = 32
DH = D // 2
L = 16
NC, NS = 2, 16
HH = H // 2


def _sc_lookup(scale_hbm, row_hbm, col_hbm, tabs_hbm, sbuf, buf, sem):
    sid = lax.axis_index("s")
    r0 = 2 * sid
    cp_s = pltpu.make_async_copy(scale_hbm, sbuf, sem)
    cp_r = pltpu.make_async_copy(row_hbm.at[pl.ds(r0, 2)], buf.at[pl.ds(0, 2)],
                                 sem)
    cp_c = pltpu.make_async_copy(col_hbm.at[pl.ds(r0, 2)], buf.at[pl.ds(2, 2)],
                                 sem)
    cp_s.start()
    cp_r.start()
    cp_c.start()
    cp_s.wait()
    cp_r.wait()
    cp_c.wait()
    s = sbuf[...]
    for j in range(4):
        for k in range(DH // L):
            buf[j, pl.ds(L * k, L)] = buf[j, pl.ds(L * k, L)] * s
    o_r = pltpu.make_async_copy(buf.at[pl.ds(0, 2)], tabs_hbm.at[pl.ds(r0, 2)],
                                sem)
    o_c = pltpu.make_async_copy(buf.at[pl.ds(2, 2)],
                                tabs_hbm.at[pl.ds(H + r0, 2)], sem)
    o_r.start()
    o_c.start()
    o_r.wait()
    o_c.wait()


def _tc_replicate(tabs_ref, out_ref, tile_ref, sems):
    c = tabs_ref[pl.ds(H, W), :]
    copies = []
    for i in range(2):
        r = tabs_ref[pl.ds(i * HH, HH), :]
        tile_ref[pl.ds(i * HH, HH), :, :DH] = jnp.broadcast_to(
            r[:, None, :], (HH, W, DH))
        tile_ref[pl.ds(i * HH, HH), :, DH:] = jnp.broadcast_to(
            c[None, :, :], (HH, W, DH))
        for b in range(B):
            cp = pltpu.make_async_copy(
                tile_ref.at[pl.ds(i * HH, HH)],
                out_ref.at[b, pl.ds(i * HH, HH)],
                sems.at[2 * b + i])
            cp.start()
            copies.append(cp)
    for cp in copies:
        cp.wait()


def kernel(row_embed, col_embed, batch_size):
    scale = (jnp.asarray(batch_size, jnp.int32) // B).astype(jnp.float32)
    scale_vec = jnp.full((L,), scale, dtype=jnp.float32)
    mesh = plsc.VectorSubcoreMesh(core_axis_name="c", subcore_axis_name="s",
                                  num_cores=1)
    lookup = functools.partial(
        pl.kernel,
        mesh=mesh,
        out_type=jax.ShapeDtypeStruct((H + W, DH), jnp.float32),
        scratch_types=[
            pltpu.VMEM((L,), jnp.float32),
            pltpu.VMEM((4, DH), jnp.float32),
            pltpu.SemaphoreType.DMA,
        ],
    )(_sc_lookup)
    tabs = lookup(scale_vec, row_embed, col_embed)
    return pl.pallas_call(
        _tc_replicate,
        in_specs=[pl.BlockSpec(memory_space=pltpu.VMEM)],
        out_specs=pl.BlockSpec(memory_space=pl.ANY),
        out_shape=jax.ShapeDtypeStruct((B, H, W, D), jnp.float32),
        scratch_shapes=[
            pltpu.VMEM((H, W, D), jnp.float32),
            pltpu.SemaphoreType.DMA((2 * B,)),
        ],
    )(tabs)
